# Initial kernel scaffold; baseline (speedup 1.0000x reference)
import jax
import jax.numpy as jnp
from jax.experimental import pallas as pl


def kernel(x, w):
    raise NotImplementedError("write your pallas kernel here")


# speedup vs baseline: 6.3097x; 6.3097x over previous
import functools

import jax
import jax.numpy as jnp
from jax import lax
from jax.experimental import pallas as pl
from jax.experimental.pallas import tpu as pltpu
from jax.experimental.pallas import tpu_sc as plsc

NC = 2
NS = 16
NW = NC * NS
CHUNK = 128


@functools.partial(jax.jit, static_argnames=("b_per_w", "n_chunks"))
def _embed(xf, w, *, b_per_w, n_chunks):
    B = xf.shape[0]
    D = w.shape[1]
    mesh = plsc.VectorSubcoreMesh(core_axis_name="c", subcore_axis_name="s")

    @functools.partial(
        pl.kernel,
        out_type=jax.ShapeDtypeStruct((B, D), jnp.float32),
        mesh=mesh,
        scratch_types=[
            pltpu.VMEM((b_per_w,), jnp.int32),
            pltpu.VMEM((CHUNK, D), jnp.float32),
            pltpu.SemaphoreType.DMA,
        ],
    )
    def body(idx_hbm, tbl_hbm, out_hbm, idx_v, rows_v, gsem):
        wid = lax.axis_index("s") * NC + lax.axis_index("c")
        base = wid * b_per_w
        pltpu.sync_copy(idx_hbm.at[pl.ds(base, b_per_w)], idx_v)

        def chunk_body(c, carry):
            off = c * CHUNK
            idx_slice = idx_v.at[pl.ds(off, CHUNK)]
            pltpu.async_copy(tbl_hbm.at[idx_slice], rows_v, gsem).wait()
            pltpu.sync_copy(rows_v, out_hbm.at[pl.ds(base + off, CHUNK)])
            return carry

        lax.fori_loop(0, n_chunks, chunk_body, 0)

    return body(xf, w)


def kernel(x, w):
    B = x.shape[0] * x.shape[1]
    D = w.shape[1]
    b_per_w = B // NW
    n_chunks = b_per_w // CHUNK
    xf = x.reshape(B).astype(jnp.int32)
    out = _embed(xf, w, b_per_w=b_per_w, n_chunks=n_chunks)
    return out.reshape(x.shape[0], x.shape[1], D)


# speedup vs baseline: 8.8486x; 1.4024x over previous
import functools

import jax
import jax.numpy as jnp
from jax import lax
from jax.experimental import pallas as pl
from jax.experimental.pallas import tpu as pltpu
from jax.experimental.pallas import tpu_sc as plsc

NC = 2
NS = 16
NW = NC * NS
CHUNK = 128


K = 2
SETS = 2


@functools.partial(jax.jit, static_argnames=("b_per_w", "n_chunks"))
def _embed(xf, w, *, b_per_w, n_chunks):
    B = xf.shape[0]
    D = w.shape[1]
    per_iter = SETS * K
    n_iters = n_chunks // per_iter
    mesh = plsc.VectorSubcoreMesh(core_axis_name="c", subcore_axis_name="s")

    @functools.partial(
        pl.kernel,
        out_type=jax.ShapeDtypeStruct((B, D), jnp.float32),
        mesh=mesh,
        scratch_types=[
            pltpu.VMEM((b_per_w,), jnp.int32),
            pltpu.VMEM((SETS, K, CHUNK, D), jnp.float32),
            pltpu.SemaphoreType.DMA,
            pltpu.SemaphoreType.DMA,
            pltpu.SemaphoreType.DMA,
            pltpu.SemaphoreType.DMA,
        ],
    )
    def body(idx_hbm, tbl_hbm, out_hbm, idx_v, rows, g0, g1, s0, s1):
        gsem = (g0, g1)
        ssem = (s0, s1)
        wid = lax.axis_index("s") * NC + lax.axis_index("c")
        base = wid * b_per_w
        pltpu.sync_copy(idx_hbm.at[pl.ds(base, b_per_w)], idx_v)

        def iter_body(i, carry):
            c0 = i * per_iter
            gd = [[None] * K for _ in range(SETS)]
            sd = [[None] * K for _ in range(SETS)]
            for s in range(SETS):
                for k in range(K):
                    off = (c0 + s * K + k) * CHUNK
                    gd[s][k] = pltpu.async_copy(
                        tbl_hbm.at[idx_v.at[pl.ds(off, CHUNK)]],
                        rows.at[s, k], gsem[s])
            for s in range(SETS):
                for k in range(K):
                    gd[s][k].wait()
                for k in range(K):
                    off = (c0 + s * K + k) * CHUNK
                    sd[s][k] = pltpu.async_copy(
                        rows.at[s, k],
                        out_hbm.at[pl.ds(base + off, CHUNK)], ssem[s])
            for s in range(SETS):
                for k in range(K):
                    sd[s][k].wait()
            return carry

        lax.fori_loop(0, n_iters, iter_body, 0)

    return body(xf, w)


def kernel(x, w):
    B = x.shape[0] * x.shape[1]
    D = w.shape[1]
    b_per_w = B // NW
    n_chunks = b_per_w // CHUNK
    xf = x.reshape(B).astype(jnp.int32)
    out = _embed(xf, w, b_per_w=b_per_w, n_chunks=n_chunks)
    return out.reshape(x.shape[0], x.shape[1], D)


# speedup vs baseline: 9.0214x; 1.0195x over previous
import functools

import jax
import jax.numpy as jnp
from jax import lax
from jax.experimental import pallas as pl
from jax.experimental.pallas import tpu as pltpu
from jax.experimental.pallas import tpu_sc as plsc

NC = 2
NS = 16
NW = NC * NS
CHUNK = 128


K = 2
SETS = 2


@functools.partial(jax.jit, static_argnames=("b_per_w", "n_chunks"))
def _embed(xf, w, *, b_per_w, n_chunks):
    B = xf.shape[0]
    D = w.shape[1]
    per_iter = SETS * K
    n_iters = n_chunks // per_iter
    mesh = plsc.VectorSubcoreMesh(core_axis_name="c", subcore_axis_name="s")

    @functools.partial(
        pl.kernel,
        out_type=jax.ShapeDtypeStruct((B, D), jnp.float32),
        mesh=mesh,
        scratch_types=[
            pltpu.VMEM((b_per_w,), jnp.int32),
            pltpu.VMEM((SETS, K, CHUNK, D), jnp.float32),
            pltpu.SemaphoreType.DMA,
            pltpu.SemaphoreType.DMA,
            pltpu.SemaphoreType.DMA,
            pltpu.SemaphoreType.DMA,
        ],
    )
    def body(idx_hbm, tbl_hbm, out_hbm, idx_v, rows, g0, g1, s0, s1):
        gsem = (g0, g1)
        ssem = (s0, s1)
        wid = lax.axis_index("s") * NC + lax.axis_index("c")
        base = wid * b_per_w
        pltpu.sync_copy(idx_hbm.at[pl.ds(base, b_per_w)], idx_v)

        def fire_gathers(c0, s):
            return [
                pltpu.async_copy(
                    tbl_hbm.at[idx_v.at[pl.ds((c0 + s * K + k) * CHUNK, CHUNK)]],
                    rows.at[s, k], gsem[s])
                for k in range(K)
            ]

        def fire_stores(c0, s):
            return [
                pltpu.async_copy(
                    rows.at[s, k],
                    out_hbm.at[pl.ds(base + (c0 + s * K + k) * CHUNK, CHUNK)],
                    ssem[s])
                for k in range(K)
            ]

        def drain_stores(s):
            for k in range(K):
                pltpu.make_async_copy(
                    rows.at[s, k], out_hbm.at[pl.ds(base, CHUNK)],
                    ssem[s]).wait()

        gd = [fire_gathers(0, s) for s in range(SETS)]
        for s in range(SETS):
            for d in gd[s]:
                d.wait()
            fire_stores(0, s)

        def iter_body(i, carry):
            c0 = i * per_iter
            gd = []
            for s in range(SETS):
                drain_stores(s)
                gd.append(fire_gathers(c0, s))
            for s in range(SETS):
                for d in gd[s]:
                    d.wait()
                fire_stores(c0, s)
            return carry

        lax.fori_loop(1, n_iters, iter_body, 0)
        for s in range(SETS):
            drain_stores(s)

    return body(xf, w)


def kernel(x, w):
    B = x.shape[0] * x.shape[1]
    D = w.shape[1]
    b_per_w = B // NW
    n_chunks = b_per_w // CHUNK
    xf = x.reshape(B).astype(jnp.int32)
    out = _embed(xf, w, b_per_w=b_per_w, n_chunks=n_chunks)
    return out.reshape(x.shape[0], x.shape[1], D)
